# baseline (speedup 1.0000x reference)
import functools

import jax
import jax.numpy as jnp
from jax import lax
from jax.experimental import pallas as pl
from jax.experimental.pallas import tpu as pltpu
from jax.experimental.pallas import tpu_sc as plsc

EMBED = 64
PACKED = 128
ROWS = 16384
COLS = 50
VOCAB_P = 500004
NC, NS = 2, 16
NW = NC * NS
CHUNK = 256
ICHUNKS = ROWS // CHUNK
UNITS = COLS * ICHUNKS
U_PER_W = UNITS // NW
NBLK = CHUNK // 16

_mesh = plsc.VectorSubcoreMesh(core_axis_name="c", subcore_axis_name="s")


@functools.partial(
    pl.kernel,
    mesh=_mesh,
    out_type=jax.ShapeDtypeStruct((COLS, EMBED, ROWS), jnp.float32),
    compiler_params=pltpu.CompilerParams(needs_layout_passes=False),
    scratch_types=[
        pltpu.VMEM((CHUNK,), jnp.int32),
        pltpu.VMEM((CHUNK,), jnp.int32),
        pltpu.VMEM((CHUNK,), jnp.int32),
        pltpu.VMEM((CHUNK,), jnp.int32),
        pltpu.VMEM((CHUNK,), jnp.int32),
        pltpu.VMEM((CHUNK,), jnp.int32),
        pltpu.VMEM((CHUNK, PACKED), jnp.float32),
        pltpu.VMEM((CHUNK, PACKED), jnp.float32),
        pltpu.VMEM((EMBED, CHUNK), jnp.float32),
        pltpu.VMEM((EMBED, CHUNK), jnp.float32),
        pltpu.SemaphoreType.DMA,
        pltpu.SemaphoreType.DMA,
    ],
)
def _sc_gather_t(idx_hbm, table_hbm, out_hbm,
                 idx0, idx1, gix0, gix1, pb0, pb1,
                 rows0, rows1, tr0, tr1, gsem0, gsem1):
    wid = lax.axis_index("s") * NC + lax.axis_index("c")
    u_base = wid * U_PER_W
    iota = lax.iota(jnp.int32, 16)

    slots = (
        (idx0, gix0, pb0, rows0, tr0, gsem0),
        (idx1, gix1, pb1, rows1, tr1, gsem1),
    )

    def start(u, slot):
        idxb, gixb, pbb, rowsb, _, gsem = slots[slot]
        j = u // ICHUNKS
        i0 = (u % ICHUNKS) * CHUNK
        pltpu.sync_copy(idx_hbm.at[j, pl.ds(i0, CHUNK)], idxb)
        for blk in range(NBLK):
            v = idxb[pl.ds(blk * 16, 16)]
            gixb[pl.ds(blk * 16, 16)] = v >> 1
            pbb[pl.ds(blk * 16, 16)] = (v & 1) * EMBED
        pltpu.async_copy(table_hbm.at[gixb], rowsb, gsem)

    def finish(u, slot):
        idxb, gixb, pbb, rowsb, trb, gsem = slots[slot]
        pltpu.make_async_copy(table_hbm.at[gixb], rowsb, gsem).wait()
        j = u // ICHUNKS
        i0 = (u % ICHUNKS) * CHUNK
        rflat = rowsb.at[0]
        for blk in range(NBLK):
            base = (iota + blk * 16) * PACKED + pbb[pl.ds(blk * 16, 16)]

            def ebody(e, carry, base=base, rflat=rflat, trb=trb, blk=blk):
                vals = plsc.load_gather(rflat, [base + e])
                trb[e, pl.ds(blk * 16, 16)] = vals
                return carry

            lax.fori_loop(0, EMBED, ebody, 0, unroll=8)
        pltpu.sync_copy(trb, out_hbm.at[j, :, pl.ds(i0, CHUNK)])

    start(u_base, 0)

    def body(g, carry):
        u = u_base + 2 * g
        start(u + 1, 1)
        finish(u, 0)

        @pl.when(g < U_PER_W // 2 - 1)
        def _():
            start(u + 2, 0)

        finish(u + 1, 1)
        return carry

    lax.fori_loop(0, U_PER_W // 2, body, 0)


def kernel(indices, table):
    idx_t = indices.T.astype(jnp.int32)
    table_r = table.reshape(VOCAB_P, PACKED)
    out_t = _sc_gather_t(idx_t, table_r)
    return out_t.transpose(2, 0, 1)


# speedup vs baseline: 1.3205x; 1.3205x over previous
import functools

import jax
import jax.numpy as jnp
from jax import lax
from jax.experimental import pallas as pl
from jax.experimental.pallas import tpu as pltpu
from jax.experimental.pallas import tpu_sc as plsc

EMBED = 64
PACKED = 128
ROWS = 16384
COLS = 50
VOCAB_P = 500004
NC, NS = 2, 16
NW = NC * NS
CHUNK = 256
ICHUNKS = ROWS // CHUNK
UNITS = COLS * ICHUNKS
U_PER_W = UNITS // NW
NBLK = CHUNK // 16

_mesh = plsc.VectorSubcoreMesh(core_axis_name="c", subcore_axis_name="s")


@functools.partial(
    pl.kernel,
    mesh=_mesh,
    out_type=jax.ShapeDtypeStruct((COLS, EMBED, ROWS), jnp.float32),
    compiler_params=pltpu.CompilerParams(needs_layout_passes=False),
    scratch_types=[
        pltpu.VMEM((CHUNK,), jnp.int32),
        pltpu.VMEM((CHUNK,), jnp.int32),
        pltpu.VMEM((CHUNK,), jnp.int32),
        pltpu.VMEM((CHUNK,), jnp.int32),
        pltpu.VMEM((CHUNK,), jnp.int32),
        pltpu.VMEM((CHUNK,), jnp.int32),
        pltpu.VMEM((CHUNK, PACKED), jnp.float32),
        pltpu.VMEM((CHUNK, PACKED), jnp.float32),
        pltpu.VMEM((EMBED, CHUNK), jnp.float32),
        pltpu.VMEM((EMBED, CHUNK), jnp.float32),
        pltpu.SemaphoreType.DMA,
        pltpu.SemaphoreType.DMA,
    ],
)
def _sc_gather_t(idx_hbm, table_hbm, out_hbm,
                 idx0, idx1, gix0, gix1, pb0, pb1,
                 rows0, rows1, tr0, tr1, gsem0, gsem1):
    wid = lax.axis_index("s") * NC + lax.axis_index("c")
    u_base = wid * U_PER_W
    iota = lax.iota(jnp.int32, 16)

    slots = (
        (idx0, gix0, pb0, rows0, tr0, gsem0),
        (idx1, gix1, pb1, rows1, tr1, gsem1),
    )

    def start(u, slot):
        idxb, gixb, pbb, rowsb, _, gsem = slots[slot]
        j = u // ICHUNKS
        i0 = (u % ICHUNKS) * CHUNK
        pltpu.sync_copy(idx_hbm.at[j, pl.ds(i0, CHUNK)], idxb)
        for blk in range(NBLK):
            v = idxb[pl.ds(blk * 16, 16)]
            gixb[pl.ds(blk * 16, 16)] = v >> 1
            pbb[pl.ds(blk * 16, 16)] = (v & 1) * EMBED
        pltpu.async_copy(table_hbm.at[gixb], rowsb, gsem)

    def finish(u, slot):
        idxb, gixb, pbb, rowsb, trb, gsem = slots[slot]
        pltpu.make_async_copy(table_hbm.at[gixb], rowsb, gsem).wait()
        j = u // ICHUNKS
        i0 = (u % ICHUNKS) * CHUNK
        for blk in range(NBLK):
            rr = iota + blk * 16
            cc = pbb[pl.ds(blk * 16, 16)]

            def e0body(e0, carry, rr=rr, cc=cc, rowsb=rowsb, trb=trb):
                rot = (iota + e0) & 15
                for e1 in range(EMBED // 16):
                    d = rot + e1 * 16
                    vals = plsc.load_gather(rowsb, [rr, cc + d])
                    plsc.store_scatter(trb, [d, rr], vals)
                return carry

            lax.fori_loop(0, 16, e0body, 0, unroll=4)
        pltpu.sync_copy(trb, out_hbm.at[j, :, pl.ds(i0, CHUNK)])

    start(u_base, 0)

    def body(g, carry):
        u = u_base + 2 * g
        start(u + 1, 1)
        finish(u, 0)

        @pl.when(g < U_PER_W // 2 - 1)
        def _():
            start(u + 2, 0)

        finish(u + 1, 1)
        return carry

    lax.fori_loop(0, U_PER_W // 2, body, 0)


def kernel(indices, table):
    idx_t = indices.T.astype(jnp.int32)
    table_r = table.reshape(VOCAB_P, PACKED)
    out_t = _sc_gather_t(idx_t, table_r)
    return out_t.transpose(2, 0, 1)


# speedup vs baseline: 1.4729x; 1.1154x over previous
import functools

import jax
import jax.numpy as jnp
from jax import lax
from jax.experimental import pallas as pl
from jax.experimental.pallas import tpu as pltpu
from jax.experimental.pallas import tpu_sc as plsc

EMBED = 64
PACKED = 128
ROWS = 16384
COLS = 50
VOCAB_P = 500004
NC, NS = 2, 16
NW = NC * NS
CHUNK = 256
ICHUNKS = ROWS // CHUNK
UNITS = COLS * ICHUNKS
U_PER_W = UNITS // NW
NBLK = CHUNK // 16
NPAIR = U_PER_W // 2

_mesh = plsc.VectorSubcoreMesh(core_axis_name="c", subcore_axis_name="s")


@functools.partial(
    pl.kernel,
    mesh=_mesh,
    out_type=jax.ShapeDtypeStruct((COLS, EMBED, ROWS), jnp.float32),
    compiler_params=pltpu.CompilerParams(needs_layout_passes=False),
    scratch_types=[
        pltpu.VMEM((CHUNK,), jnp.int32),
        pltpu.VMEM((CHUNK,), jnp.int32),
        pltpu.VMEM((CHUNK,), jnp.int32),
        pltpu.VMEM((CHUNK,), jnp.int32),
        pltpu.VMEM((CHUNK,), jnp.int32),
        pltpu.VMEM((CHUNK,), jnp.int32),
        pltpu.VMEM((CHUNK, PACKED), jnp.float32),
        pltpu.VMEM((CHUNK, PACKED), jnp.float32),
        pltpu.VMEM((EMBED, CHUNK), jnp.float32),
        pltpu.VMEM((EMBED, CHUNK), jnp.float32),
        pltpu.SemaphoreType.DMA,
        pltpu.SemaphoreType.DMA,
        pltpu.SemaphoreType.DMA,
        pltpu.SemaphoreType.DMA,
        pltpu.SemaphoreType.DMA,
        pltpu.SemaphoreType.DMA,
    ],
)
def _sc_gather_t(idx_hbm, table_hbm, out_hbm,
                 idx0, idx1, gix0, gix1, pb0, pb1,
                 rows0, rows1, tr0, tr1,
                 isem0, isem1, gsem0, gsem1, osem0, osem1):
    wid = lax.axis_index("s") * NC + lax.axis_index("c")
    u_base = wid * U_PER_W
    iota = lax.iota(jnp.int32, 16)

    slots = (
        (idx0, gix0, pb0, rows0, tr0, isem0, gsem0, osem0),
        (idx1, gix1, pb1, rows1, tr1, isem1, gsem1, osem1),
    )

    def uoff(u):
        return u // ICHUNKS, (u % ICHUNKS) * CHUNK

    def idx_load(u, s):
        idxb, _, _, _, _, isem, _, _ = slots[s]
        j, i0 = uoff(u)
        pltpu.async_copy(idx_hbm.at[j, pl.ds(i0, CHUNK)], idxb, isem)

    def prep(u, s):
        idxb, gixb, pbb, rowsb, _, isem, gsem, _ = slots[s]
        j, i0 = uoff(u)
        pltpu.make_async_copy(idx_hbm.at[j, pl.ds(i0, CHUNK)], idxb, isem).wait()
        for blk in range(NBLK):
            v = idxb[pl.ds(blk * 16, 16)]
            gixb[pl.ds(blk * 16, 16)] = v >> 1
            pbb[pl.ds(blk * 16, 16)] = (v & 1) * EMBED
        pltpu.async_copy(table_hbm.at[gixb], rowsb, gsem)

    def trans(u, s, wait_out):
        _, gixb, pbb, rowsb, trb, _, gsem, osem = slots[s]
        j, i0 = uoff(u)
        pltpu.make_async_copy(table_hbm.at[gixb], rowsb, gsem).wait()
        if wait_out is not None:
            @pl.when(wait_out)
            def _():
                pltpu.make_async_copy(
                    trb, out_hbm.at[j, :, pl.ds(i0, CHUNK)], osem
                ).wait()
        for blk in range(NBLK):
            rr = iota + blk * 16
            cc = pbb[pl.ds(blk * 16, 16)]

            def e0body(e0, carry, rr=rr, cc=cc, rowsb=rowsb, trb=trb):
                rot = (iota + e0) & 15
                for e1 in range(EMBED // 16):
                    d = rot + e1 * 16
                    vals = plsc.load_gather(rowsb, [rr, cc + d])
                    plsc.store_scatter(trb, [d, rr], vals)
                return carry

            lax.fori_loop(0, 16, e0body, 0, unroll=4)
        pltpu.async_copy(trb, out_hbm.at[j, :, pl.ds(i0, CHUNK)], osem)

    idx_load(u_base, 0)
    idx_load(u_base + 1, 1)
    prep(u_base, 0)

    def body(gp, carry):
        u = u_base + 2 * gp
        more = gp < NPAIR - 1

        @pl.when(more)
        def _():
            idx_load(u + 2, 0)

        prep(u + 1, 1)
        trans(u, 0, wait_out=gp >= 1)

        @pl.when(more)
        def _():
            idx_load(u + 3, 1)
            prep(u + 2, 0)

        trans(u + 1, 1, wait_out=gp >= 1)
        return carry

    lax.fori_loop(0, NPAIR, body, 0)

    j0, i00 = uoff(u_base + U_PER_W - 2)
    pltpu.make_async_copy(tr0, out_hbm.at[j0, :, pl.ds(i00, CHUNK)], osem0).wait()
    j1, i01 = uoff(u_base + U_PER_W - 1)
    pltpu.make_async_copy(tr1, out_hbm.at[j1, :, pl.ds(i01, CHUNK)], osem1).wait()


def kernel(indices, table):
    idx_t = indices.T.astype(jnp.int32)
    table_r = table.reshape(VOCAB_P, PACKED)
    out_t = _sc_gather_t(idx_t, table_r)
    return out_t.transpose(2, 0, 1)


# speedup vs baseline: 2.2144x; 1.5034x over previous
import functools

import jax
import jax.numpy as jnp
from jax import lax
from jax.experimental import pallas as pl
from jax.experimental.pallas import tpu as pltpu
from jax.experimental.pallas import tpu_sc as plsc

EMBED = 64
PACKED = 128
ROWS = 16384
COLS = 50
VOCAB_P = 500004
NC, NS = 2, 16
NW = NC * NS
CHUNK = 256
ICHUNKS = ROWS // CHUNK
UNITS = COLS * ICHUNKS
U_PER_W = UNITS // NW
NBLK = CHUNK // 16
NPAIR = U_PER_W // 2

_mesh = plsc.VectorSubcoreMesh(core_axis_name="c", subcore_axis_name="s")


@functools.partial(
    pl.kernel,
    mesh=_mesh,
    out_type=jax.ShapeDtypeStruct((COLS, EMBED, ROWS), jnp.float32),
    compiler_params=pltpu.CompilerParams(needs_layout_passes=False),
    scratch_types=[
        pltpu.VMEM((CHUNK,), jnp.int32),
        pltpu.VMEM((CHUNK,), jnp.int32),
        pltpu.VMEM((CHUNK,), jnp.int32),
        pltpu.VMEM((CHUNK,), jnp.int32),
        pltpu.VMEM((CHUNK,), jnp.int32),
        pltpu.VMEM((CHUNK,), jnp.int32),
        pltpu.VMEM((CHUNK, PACKED), jnp.float32),
        pltpu.VMEM((CHUNK, PACKED), jnp.float32),
        pltpu.VMEM((EMBED, CHUNK), jnp.float32),
        pltpu.VMEM((EMBED, CHUNK), jnp.float32),
        pltpu.SemaphoreType.DMA,
        pltpu.SemaphoreType.DMA,
        pltpu.SemaphoreType.DMA,
        pltpu.SemaphoreType.DMA,
        pltpu.SemaphoreType.DMA,
        pltpu.SemaphoreType.DMA,
    ],
)
def _sc_gather_t(idx_hbm, table_hbm, out_hbm,
                 idx0, idx1, gix0, gix1, pb0, pb1,
                 rows0, rows1, tr0, tr1,
                 isem0, isem1, gsem0, gsem1, osem0, osem1):
    wid = lax.axis_index("s") * NC + lax.axis_index("c")
    u_base = wid * U_PER_W
    iota = lax.iota(jnp.int32, 16)

    slots = (
        (idx0, gix0, pb0, rows0, tr0, isem0, gsem0, osem0),
        (idx1, gix1, pb1, rows1, tr1, isem1, gsem1, osem1),
    )

    def uoff(u):
        return u // ICHUNKS, (u % ICHUNKS) * CHUNK

    def idx_load(u, s):
        idxb, _, _, _, _, isem, _, _ = slots[s]
        j, i0 = uoff(u)
        pltpu.async_copy(idx_hbm.at[j, pl.ds(i0, CHUNK)], idxb, isem)

    def prep(u, s):
        idxb, gixb, pbb, rowsb, _, isem, gsem, _ = slots[s]
        j, i0 = uoff(u)
        pltpu.make_async_copy(idx_hbm.at[j, pl.ds(i0, CHUNK)], idxb, isem).wait()
        for blk in range(NBLK):
            v = idxb[pl.ds(blk * 16, 16)]
            gixb[pl.ds(blk * 16, 16)] = v >> 1
            pbb[pl.ds(blk * 16, 16)] = (v & 1) * EMBED
        pltpu.async_copy(table_hbm.at[gixb], rowsb, gsem)

    def trans(u, s, wait_out):
        _, gixb, pbb, rowsb, trb, _, gsem, osem = slots[s]
        j, i0 = uoff(u)
        pltpu.make_async_copy(table_hbm.at[gixb], rowsb, gsem).wait()
        if wait_out is not None:
            @pl.when(wait_out)
            def _():
                pltpu.make_async_copy(
                    trb, out_hbm.at[j, :, pl.ds(i0, CHUNK)], osem
                ).wait()
        for blk in range(NBLK):
            rr = iota + blk * 16
            cc = pbb[pl.ds(blk * 16, 16)]

            def e0body(e0, rr=rr, cc=cc, rowsb=rowsb, trb=trb):
                rot = (iota + e0) & 15
                for e1 in range(EMBED // 16):
                    d = rot + e1 * 16
                    vals = plsc.load_gather(rowsb, [rr, cc + d])
                    plsc.store_scatter(trb, [d, rr], vals)

            plsc.parallel_loop(0, 16, unroll=4)(e0body)
        pltpu.async_copy(trb, out_hbm.at[j, :, pl.ds(i0, CHUNK)], osem)

    idx_load(u_base, 0)
    idx_load(u_base + 1, 1)
    prep(u_base, 0)

    def body(gp, carry):
        u = u_base + 2 * gp
        more = gp < NPAIR - 1

        @pl.when(more)
        def _():
            idx_load(u + 2, 0)

        prep(u + 1, 1)
        trans(u, 0, wait_out=gp >= 1)

        @pl.when(more)
        def _():
            idx_load(u + 3, 1)
            prep(u + 2, 0)

        trans(u + 1, 1, wait_out=gp >= 1)
        return carry

    lax.fori_loop(0, NPAIR, body, 0)

    j0, i00 = uoff(u_base + U_PER_W - 2)
    pltpu.make_async_copy(tr0, out_hbm.at[j0, :, pl.ds(i00, CHUNK)], osem0).wait()
    j1, i01 = uoff(u_base + U_PER_W - 1)
    pltpu.make_async_copy(tr1, out_hbm.at[j1, :, pl.ds(i01, CHUNK)], osem1).wait()


def kernel(indices, table):
    idx_t = indices.T.astype(jnp.int32)
    table_r = table.reshape(VOCAB_P, PACKED)
    out_t = _sc_gather_t(idx_t, table_r)
    return out_t.transpose(2, 0, 1)


# speedup vs baseline: 3.9524x; 1.7848x over previous
import functools

import jax
import jax.numpy as jnp
from jax import lax
from jax.experimental import pallas as pl
from jax.experimental.pallas import tpu as pltpu
from jax.experimental.pallas import tpu_sc as plsc

EMBED = 64
PACKED = 128
ROWS = 16384
COLS = 50
VOCAB_P = 500004
NC, NS = 2, 16
NW = NC * NS
CHUNK = 256
ICHUNKS = ROWS // CHUNK
UNITS = COLS * ICHUNKS
U_PER_W = UNITS // NW
NBLK = CHUNK // 16
NPAIR = U_PER_W // 2

_mesh = plsc.VectorSubcoreMesh(core_axis_name="c", subcore_axis_name="s")

VOCAB = 1000008
PCH = 7812
PTAIL = VOCAB - PCH * PACKED
PK_MAX = 245
PK_EXTRA = PCH - 244 * NW


@functools.partial(
    pl.kernel,
    mesh=_mesh,
    out_type=jax.ShapeDtypeStruct((VOCAB_P, PACKED), jnp.float32),
    compiler_params=pltpu.CompilerParams(needs_layout_passes=False),
    scratch_types=[
        pltpu.VMEM((EMBED, PACKED), jnp.float32),
        pltpu.VMEM((EMBED, PACKED), jnp.float32),
        pltpu.VMEM((EMBED, PACKED), jnp.float32),
        pltpu.VMEM((EMBED, PACKED), jnp.float32),
        pltpu.SemaphoreType.DMA,
        pltpu.SemaphoreType.DMA,
        pltpu.SemaphoreType.DMA,
        pltpu.SemaphoreType.DMA,
    ],
)
def _sc_pack(tab_hbm, tail_hbm, tpack_hbm, sb0, sb1, tb0, tb1,
             isem0, isem1, osem0, osem1):
    wid = lax.axis_index("s") * NC + lax.axis_index("c")
    nk = jnp.where(wid < PK_EXTRA, PK_MAX, PK_MAX - 1)
    iota = lax.iota(jnp.int32, 16)

    slots = ((sb0, tb0, isem0, osem0), (sb1, tb1, isem1, osem1))

    def col0_of(k):
        return (wid + NW * k) * PACKED

    def load(k, s):
        sb, _, isem, _ = slots[s]
        pltpu.async_copy(tab_hbm.at[:, pl.ds(col0_of(k), PACKED)], sb, isem)

    def work(k, s, wait_out):
        sb, tb, isem, osem = slots[s]
        col0 = col0_of(k)
        row0 = (wid + NW * k) * (PACKED // 2)
        pltpu.make_async_copy(
            tab_hbm.at[:, pl.ds(col0, PACKED)], sb, isem
        ).wait()

        @pl.when(wait_out)
        def _():
            pltpu.make_async_copy(
                tb, tpack_hbm.at[pl.ds(0, EMBED), :], osem
            ).wait()

        def sbody(s_rot, sb=sb, tb=tb):
            rot = (iota + s_rot) & 15
            for cb in range(PACKED // 16):
                ci = cb * 16 + iota
                ci64 = ci * EMBED
                for eb in range(EMBED // 16):
                    srows = eb * 16 + rot
                    vals = plsc.load_gather(sb, [srows, ci])
                    dest = ci64 + srows
                    plsc.store_scatter(tb, [dest >> 7, dest & 127], vals)

        plsc.parallel_loop(0, 16, unroll=2)(sbody)
        pltpu.async_copy(tb, tpack_hbm.at[pl.ds(row0, EMBED), :], osem)

    load(0, 0)

    @pl.when(nk > 1)
    def _():
        load(1, 1)

    def body(kp, carry):
        k = 2 * kp

        @pl.when(k + 2 < nk)
        def _():
            load(k + 2, 0)

        @pl.when(k < nk)
        def _():
            work(k, 0, wait_out=kp >= 1)

        @pl.when(k + 3 < nk)
        def _():
            load(k + 3, 1)

        @pl.when(k + 1 < nk)
        def _():
            work(k + 1, 1, wait_out=kp >= 1)

        return carry

    lax.fori_loop(0, (PK_MAX + 1) // 2, body, 0)

    pltpu.make_async_copy(tb0, tpack_hbm.at[pl.ds(0, EMBED), :], osem0).wait()
    pltpu.make_async_copy(tb1, tpack_hbm.at[pl.ds(0, EMBED), :], osem1).wait()

    @pl.when(wid == NW - 1)
    def _():
        pltpu.sync_copy(
            tail_hbm,
            tpack_hbm.at[pl.ds(VOCAB_P - PTAIL // 2, PTAIL // 2), :],
        )


@functools.partial(
    pl.kernel,
    mesh=_mesh,
    out_type=jax.ShapeDtypeStruct((COLS, EMBED, ROWS), jnp.float32),
    compiler_params=pltpu.CompilerParams(needs_layout_passes=False),
    scratch_types=[
        pltpu.VMEM((CHUNK,), jnp.int32),
        pltpu.VMEM((CHUNK,), jnp.int32),
        pltpu.VMEM((CHUNK,), jnp.int32),
        pltpu.VMEM((CHUNK,), jnp.int32),
        pltpu.VMEM((CHUNK,), jnp.int32),
        pltpu.VMEM((CHUNK,), jnp.int32),
        pltpu.VMEM((CHUNK, PACKED), jnp.float32),
        pltpu.VMEM((CHUNK, PACKED), jnp.float32),
        pltpu.VMEM((EMBED, CHUNK), jnp.float32),
        pltpu.VMEM((EMBED, CHUNK), jnp.float32),
        pltpu.SemaphoreType.DMA,
        pltpu.SemaphoreType.DMA,
        pltpu.SemaphoreType.DMA,
        pltpu.SemaphoreType.DMA,
        pltpu.SemaphoreType.DMA,
        pltpu.SemaphoreType.DMA,
    ],
)
def _sc_gather_t(idx_hbm, table_hbm, out_hbm,
                 idx0, idx1, gix0, gix1, pb0, pb1,
                 rows0, rows1, tr0, tr1,
                 isem0, isem1, gsem0, gsem1, osem0, osem1):
    wid = lax.axis_index("s") * NC + lax.axis_index("c")
    u_base = wid * U_PER_W
    iota = lax.iota(jnp.int32, 16)

    slots = (
        (idx0, gix0, pb0, rows0, tr0, isem0, gsem0, osem0),
        (idx1, gix1, pb1, rows1, tr1, isem1, gsem1, osem1),
    )

    def uoff(u):
        return u // ICHUNKS, (u % ICHUNKS) * CHUNK

    def idx_load(u, s):
        idxb, _, _, _, _, isem, _, _ = slots[s]
        j, i0 = uoff(u)
        pltpu.async_copy(idx_hbm.at[j, pl.ds(i0, CHUNK)], idxb, isem)

    def prep(u, s):
        idxb, gixb, pbb, rowsb, _, isem, gsem, _ = slots[s]
        j, i0 = uoff(u)
        pltpu.make_async_copy(idx_hbm.at[j, pl.ds(i0, CHUNK)], idxb, isem).wait()
        for blk in range(NBLK):
            v = idxb[pl.ds(blk * 16, 16)]
            gixb[pl.ds(blk * 16, 16)] = v >> 1
            pbb[pl.ds(blk * 16, 16)] = (v & 1) * EMBED
        pltpu.async_copy(table_hbm.at[gixb], rowsb, gsem)

    def trans(u, s, wait_out):
        _, gixb, pbb, rowsb, trb, _, gsem, osem = slots[s]
        j, i0 = uoff(u)
        pltpu.make_async_copy(table_hbm.at[gixb], rowsb, gsem).wait()
        if wait_out is not None:
            @pl.when(wait_out)
            def _():
                pltpu.make_async_copy(
                    trb, out_hbm.at[j, :, pl.ds(i0, CHUNK)], osem
                ).wait()
        for blk in range(NBLK):
            rr = iota + blk * 16
            cc = pbb[pl.ds(blk * 16, 16)]

            def e0body(e0, rr=rr, cc=cc, rowsb=rowsb, trb=trb):
                rot = (iota + e0) & 15
                for e1 in range(EMBED // 16):
                    d = rot + e1 * 16
                    vals = plsc.load_gather(rowsb, [rr, cc + d])
                    plsc.store_scatter(trb, [d, rr], vals)

            plsc.parallel_loop(0, 16, unroll=4)(e0body)
        pltpu.async_copy(trb, out_hbm.at[j, :, pl.ds(i0, CHUNK)], osem)

    idx_load(u_base, 0)
    idx_load(u_base + 1, 1)
    prep(u_base, 0)

    def body(gp, carry):
        u = u_base + 2 * gp
        more = gp < NPAIR - 1

        @pl.when(more)
        def _():
            idx_load(u + 2, 0)

        prep(u + 1, 1)
        trans(u, 0, wait_out=gp >= 1)

        @pl.when(more)
        def _():
            idx_load(u + 3, 1)
            prep(u + 2, 0)

        trans(u + 1, 1, wait_out=gp >= 1)
        return carry

    lax.fori_loop(0, NPAIR, body, 0)

    j0, i00 = uoff(u_base + U_PER_W - 2)
    pltpu.make_async_copy(tr0, out_hbm.at[j0, :, pl.ds(i00, CHUNK)], osem0).wait()
    j1, i01 = uoff(u_base + U_PER_W - 1)
    pltpu.make_async_copy(tr1, out_hbm.at[j1, :, pl.ds(i01, CHUNK)], osem1).wait()


def kernel(indices, table):
    idx_t = indices.T.astype(jnp.int32)
    table_t = table.T
    tail_rm = table[PCH * PACKED:, :].reshape(PTAIL // 2, PACKED)
    table_r = _sc_pack(table_t, tail_rm)
    out_t = _sc_gather_t(idx_t, table_r)
    return out_t.transpose(2, 0, 1)
